# baseline (speedup 1.0000x reference)
import functools

import jax
import jax.numpy as jnp
from jax import lax
from jax.experimental import pallas as pl
from jax.experimental.pallas import tpu as pltpu
from jax.experimental.pallas import tpu_sc as plsc

N = 10000
E = 320000
D = 128
C = 64
L = 24
NCLS = 10

NC = 2
NS = 16
NW = NC * NS
LN = 16

NP = 10240
RPW = NP // NW
CH = 128
DUMP = 370
ACCR = 384

BLK = 2048
NBLK = 157
EPS = NBLK * BLK
SENT = 0x7FFF0000
CAPM = 22528
NVEC = CAPM // LN

NB = 4
LA = 2

_mesh = plsc.VectorSubcoreMesh(
    core_axis_name="c", subcore_axis_name="s", num_cores=NC, num_subcores=NS)

_SC_PARAMS = pltpu.CompilerParams(use_tc_tiling_on_sc=False,
                                  needs_layout_passes=False)


@functools.partial(
    pl.kernel,
    out_type=(
        jax.ShapeDtypeStruct((NW, 3 * CAPM), jnp.int32),
        jax.ShapeDtypeStruct((NW, 16), jnp.int32),
    ),
    mesh=_mesh,
    scratch_types=[
        pltpu.VMEM((2, BLK), jnp.int32),
        pltpu.VMEM((2, BLK), jnp.int32),
        pltpu.VMEM((2, BLK), jnp.int32),
        pltpu.VMEM((CAPM,), jnp.int32),
        pltpu.VMEM((CAPM,), jnp.int32),
        pltpu.VMEM((CAPM,), jnp.int32),
        pltpu.VMEM((ACCR,), jnp.float32),
        pltpu.VMEM((16,), jnp.int32),
        pltpu.VMEM((16,), jnp.float32),
        pltpu.SemaphoreType.DMA((2,)),
    ],
    compiler_params=_SC_PARAMS,
)
def _sc_bucket(srcl, dstl, srcg, dstg, wgi, scal_hbm,
               rec_out, cnt_out, sblk, dblk, wblk, s_stg, d_stg, w_stg,
               degv, cbuf, agbuf, bsem):
    cid = lax.axis_index("c")
    sid = lax.axis_index("s")
    wid = sid * NC + cid
    lo = wid * RPW
    hi = lo + RPW
    iota = lax.iota(jnp.int32, LN)
    ones16 = jnp.ones((LN,), jnp.float32)

    pltpu.sync_copy(scal_hbm, agbuf)
    ag = agbuf[pl.ds(0, LN)]
    alpha = ag[0]
    gamma = ag[1]
    a1 = 1.0 - alpha

    def _fill(t, carry):
        s_stg[pl.ds(t * LN, LN)] = jnp.zeros((LN,), jnp.int32)
        d_stg[pl.ds(t * LN, LN)] = jnp.zeros((LN,), jnp.int32) + DUMP
        w_stg[pl.ds(t * LN, LN)] = jnp.zeros((LN,), jnp.int32)
        return carry
    lax.fori_loop(0, NVEC, _fill, 0)

    def _zdeg(t, carry):
        degv[pl.ds(t * LN, LN)] = jnp.zeros((LN,), jnp.float32)
        return carry
    lax.fori_loop(0, ACCR // LN, _zdeg, 0)

    def _issue(sref, dref, wref, p, b, weighted):
        off = b * BLK
        pltpu.async_copy(sref.at[pl.ds(off, BLK)], sblk.at[p], bsem.at[p])
        pltpu.async_copy(dref.at[pl.ds(off, BLK)], dblk.at[p], bsem.at[p])
        if weighted:
            pltpu.async_copy(wref.at[pl.ds(off, BLK)], wblk.at[p],
                             bsem.at[p])

    def _scan_set(sref, dref, wref, wptr0, weighted):
        for p in range(2):
            _issue(sref, dref, wref, p, p, weighted)

        def _grp(g, wptr):
            for p in range(2):
                b = g * 2 + p

                @pl.when(b < NBLK)
                def _w():
                    for _ in range(3 if weighted else 2):
                        pltpu.make_async_copy(
                            sref.at[pl.ds(0, BLK)], sblk.at[p],
                            bsem.at[p]).wait()

                valid = b < NBLK

                def _vec(kk, wp):
                    d16 = dblk[p, pl.ds(kk * LN, LN)]
                    mask = jnp.logical_and(
                        jnp.logical_and(d16 >= lo, d16 < hi), valid)
                    mi = mask.astype(jnp.int32)
                    cnt = plsc.all_reduce_population_count(mask)[0]
                    dl16 = jnp.clip(d16 - lo, 0, RPW - 1)
                    ranks = plsc.cumsum(mi) - mi
                    wp2 = jnp.minimum(wp, CAPM - LN)
                    pos = wp2 + ranks
                    s16 = sblk[p, pl.ds(kk * LN, LN)]
                    plsc.store_scatter(s_stg, [pos], s16, mask=mask)
                    plsc.store_scatter(d_stg, [pos], dl16, mask=mask)
                    if weighted:
                        w16 = wblk[p, pl.ds(kk * LN, LN)]
                        plsc.store_scatter(w_stg, [pos], w16, mask=mask)
                    return wp + cnt
                wptr2 = lax.fori_loop(0, BLK // LN, _vec, wptr)

                @pl.when(b + 2 < NBLK)
                def _i():
                    _issue(sref, dref, wref, p, b + 2, weighted)
                wptr = wptr2
            return wptr
        return lax.fori_loop(0, (NBLK + 1) // 2, _grp, wptr0)

    cl = _scan_set(srcl, dstl, wgi, 0, weighted=False)
    cl = jnp.minimum(cl, CAPM)
    cm = _scan_set(srcg, dstg, wgi, cl, weighted=True)
    cm = jnp.minimum(cm, CAPM)

    lane0 = iota < 1

    def _deg(t, carry):
        base = t * LN
        dl16 = d_stg[pl.ds(base, LN)]
        for ii in range(LN):
            ridx = jnp.zeros((LN,), jnp.int32) + dl16[ii]
            m = jnp.logical_and(lane0, base + ii < cl)
            plsc.addupdate_scatter(degv, [ridx], ones16, mask=m)
        return carry
    lax.fori_loop(0, NVEC, _deg, 0)

    def _inv(t, carry):
        dv = degv[pl.ds(t * LN, LN)]
        degv[pl.ds(t * LN, LN)] = a1 / jnp.maximum(dv, 1.0)
        return carry
    lax.fori_loop(0, ACCR // LN, _inv, 0)

    a1g = a1 * gamma

    def _rw(t, carry):
        base = t * LN
        dl16 = d_stg[pl.ds(base, LN)]
        wv = plsc.bitcast(w_stg[pl.ds(base, LN)], jnp.float32)
        winv = plsc.load_gather(degv, [jnp.clip(dl16, 0, ACCR - 1)])
        islocal = (base + iota) < cl
        wnew = jnp.where(islocal, winv, wv * a1g)
        w_stg[pl.ds(base, LN)] = plsc.bitcast(wnew, jnp.int32)
        return carry
    lax.fori_loop(0, NVEC, _rw, 0)

    cbuf[pl.ds(0, LN)] = jnp.zeros((LN,), jnp.int32) + cm
    pltpu.sync_copy(cbuf, cnt_out.at[wid])
    pltpu.sync_copy(s_stg, rec_out.at[wid, pl.ds(0, CAPM)])
    pltpu.sync_copy(d_stg, rec_out.at[wid, pl.ds(CAPM, CAPM)])
    pltpu.sync_copy(w_stg, rec_out.at[wid, pl.ds(2 * CAPM, CAPM)])


@functools.partial(
    pl.kernel,
    out_type=jax.ShapeDtypeStruct((NP, C), jnp.float32),
    mesh=_mesh,
    scratch_types=[
        pltpu.VMEM((ACCR, C), jnp.float32),
        pltpu.VMEM((CAPM,), jnp.int32),
        pltpu.VMEM((CAPM,), jnp.int32),
        pltpu.VMEM((CAPM,), jnp.int32),
        pltpu.VMEM((NB, CH, C), jnp.float32),
        pltpu.VMEM((16,), jnp.int32),
        pltpu.SemaphoreType.DMA((NB,)),
    ],
    compiler_params=_SC_PARAMS,
)
def _sc_aggregate(h_hbm, rec_hbm, cnt_hbm,
                  out_hbm, acc, sbuf, dbuf, wbuf, rows, cntb, gsem):
    cid = lax.axis_index("c")
    sid = lax.axis_index("s")
    wid = sid * NC + cid
    iota = lax.iota(jnp.int32, LN)
    cols = [iota + j * LN for j in range(C // LN)]

    def _z(r, carry):
        for j in range(C // LN):
            acc[r, pl.ds(j * LN, LN)] = jnp.zeros((LN,), jnp.float32)
        return carry
    lax.fori_loop(0, ACCR, _z, 0)

    pltpu.sync_copy(cnt_hbm.at[wid], cntb)
    cnt = cntb[pl.ds(0, LN)][0]
    nch = (cnt + CH - 1) // CH
    pltpu.sync_copy(rec_hbm.at[wid, pl.ds(0, CAPM)], sbuf)
    pltpu.sync_copy(rec_hbm.at[wid, pl.ds(CAPM, CAPM)], dbuf)
    pltpu.sync_copy(rec_hbm.at[wid, pl.ds(2 * CAPM, CAPM)], wbuf)

    ngroups = (nch + LA + NB - 1) // NB

    def _grp(g, carry):
        for u in range(NB):
            i = g * NB + u

            @pl.when(i < nch)
            def _issue():
                pltpu.async_copy(
                    h_hbm.at[sbuf.at[pl.ds(i * CH, CH)]],
                    rows.at[u], gsem.at[u])

            k = i - LA
            bu = (u + LA) % NB

            @pl.when(jnp.logical_and(k >= 0, k < nch))
            def _process():
                pltpu.make_async_copy(
                    h_hbm.at[sbuf.at[pl.ds(0, CH)]],
                    rows.at[bu], gsem.at[bu]).wait()

                def _proc(kk, c3):
                    base = k * CH + kk * LN
                    dl16 = jnp.clip(dbuf[pl.ds(base, LN)], 0, ACCR - 1)
                    w16 = plsc.bitcast(wbuf[pl.ds(base, LN)], jnp.float32)
                    for ii in range(LN):
                        ridx = jnp.zeros((LN,), jnp.int32) + dl16[ii]
                        r = kk * LN + ii
                        w = w16[ii]
                        for j in range(C // LN):
                            v = rows[bu, r, pl.ds(j * LN, LN)] * w
                            plsc.addupdate_scatter(acc, [ridx, cols[j]], v)
                    return c3
                lax.fori_loop(0, CH // LN, _proc, 0)
        return carry
    lax.fori_loop(0, ngroups, _grp, 0)

    pltpu.sync_copy(acc.at[pl.ds(0, RPW)],
                    out_hbm.at[pl.ds(wid * RPW, RPW)])


def _pre_body(x_ref, w_ref, b_ref, sc_ref, h0_ref, ax0_ref):
    alpha = sc_ref[0, 0]
    h0 = jnp.dot(x_ref[...], w_ref[...],
                 preferred_element_type=jnp.float32) + b_ref[...]
    h0_ref[...] = h0
    ax0_ref[...] = alpha * h0


_tc_pre = pl.pallas_call(
    _pre_body,
    out_shape=(
        jax.ShapeDtypeStruct((NP, C), jnp.float32),
        jax.ShapeDtypeStruct((NP, C), jnp.float32),
    ),
    in_specs=[
        pl.BlockSpec(memory_space=pltpu.VMEM),
        pl.BlockSpec(memory_space=pltpu.VMEM),
        pl.BlockSpec(memory_space=pltpu.VMEM),
        pl.BlockSpec(memory_space=pltpu.SMEM),
    ],
)


def _layer_body(p_ref, ax0_ref, w_ref, b_ref, h_ref):
    hp = p_ref[...] + ax0_ref[...]
    h = jnp.dot(hp, w_ref[...], preferred_element_type=jnp.float32) + b_ref[...]
    h_ref[...] = jnp.maximum(h, 0.0)


_tc_layer = pl.pallas_call(
    _layer_body,
    out_shape=jax.ShapeDtypeStruct((NP, C), jnp.float32),
)


def _out_body(h_ref, w_ref, b_ref, o_ref):
    logits = jnp.dot(h_ref[:N], w_ref[...],
                     preferred_element_type=jnp.float32) + b_ref[...]
    m = jnp.max(logits, axis=1, keepdims=True)
    z = logits - m
    o_ref[...] = z - jnp.log(jnp.sum(jnp.exp(z), axis=1, keepdims=True))


_tc_out = pl.pallas_call(
    _out_body,
    out_shape=jax.ShapeDtypeStruct((N, NCLS), jnp.float32),
)


def kernel(x, edge_index, edge_index_global, edge_weight_global,
           W_in, b_in, W_layers, b_layers, W_out, b_out, alpha, gamma):
    pad = EPS - E
    sl2 = jnp.pad(edge_index[0], (0, pad))
    dl2 = jnp.pad(edge_index[1], (0, pad), constant_values=SENT)
    sg2 = jnp.pad(edge_index_global[0], (0, pad))
    dg2 = jnp.pad(edge_index_global[1], (0, pad), constant_values=SENT)
    wgi = lax.bitcast_convert_type(
        jnp.pad(edge_weight_global, (0, pad)), jnp.int32)
    scal16 = jnp.pad(jnp.stack([alpha, gamma]), (0, 14))
    rec, counts = _sc_bucket(sl2, dl2, sg2, dg2, wgi, scal16)

    xp = jnp.pad(x, ((0, NP - N), (0, 0)))
    scal = jnp.stack([alpha, gamma]).reshape(1, 2)
    h0, ax0 = _tc_pre(xp, W_in, b_in.reshape(1, C), scal)

    h = h0
    for i in range(L - 2):
        part = _sc_aggregate(h, rec, counts)
        h = _tc_layer(part, ax0, W_layers[i], b_layers[i].reshape(1, C))

    return _tc_out(h, W_out, b_out.reshape(1, NCLS))


# speedup vs baseline: 1.0759x; 1.0759x over previous
import functools

import jax
import jax.numpy as jnp
from jax import lax
from jax.experimental import pallas as pl
from jax.experimental.pallas import tpu as pltpu
from jax.experimental.pallas import tpu_sc as plsc

N = 10000
E = 320000
D = 128
C = 64
L = 24
NCLS = 10

NC = 2
NS = 16
NW = NC * NS
LN = 16

NP = 10240
RPW = NP // NW
CH = 128
DUMP = 370
ACCR = 384

BLK = 2048
NBLK = 157
EPS = NBLK * BLK
SENT = 0x7FFF0000
CAPM = 22528
NVEC = CAPM // LN

NB = 4
LA = 2

_mesh = plsc.VectorSubcoreMesh(
    core_axis_name="c", subcore_axis_name="s", num_cores=NC, num_subcores=NS)

_SC_PARAMS = pltpu.CompilerParams(use_tc_tiling_on_sc=False,
                                  needs_layout_passes=False)


@functools.partial(
    pl.kernel,
    out_type=(
        jax.ShapeDtypeStruct((NW, 3 * CAPM), jnp.int32),
        jax.ShapeDtypeStruct((NW, 16), jnp.int32),
        jax.ShapeDtypeStruct((NW, ACCR), jnp.float32),
    ),
    mesh=_mesh,
    scratch_types=[
        pltpu.VMEM((2, BLK), jnp.int32),
        pltpu.VMEM((2, BLK), jnp.int32),
        pltpu.VMEM((2, BLK), jnp.int32),
        pltpu.VMEM((CAPM,), jnp.int32),
        pltpu.VMEM((CAPM,), jnp.int32),
        pltpu.VMEM((CAPM,), jnp.int32),
        pltpu.VMEM((ACCR,), jnp.float32),
        pltpu.VMEM((16,), jnp.int32),
        pltpu.VMEM((16,), jnp.float32),
        pltpu.SemaphoreType.DMA((2,)),
    ],
    compiler_params=_SC_PARAMS,
)
def _sc_bucket(srcl, dstl, srcg, dstg, wgi, scal_hbm,
               rec_out, cnt_out, inv_out, sblk, dblk, wblk, s_stg, d_stg,
               w_stg, degv, cbuf, agbuf, bsem):
    cid = lax.axis_index("c")
    sid = lax.axis_index("s")
    wid = sid * NC + cid
    lo = wid * RPW
    hi = lo + RPW
    iota = lax.iota(jnp.int32, LN)
    ones16 = jnp.ones((LN,), jnp.float32)

    pltpu.sync_copy(scal_hbm, agbuf)
    ag = agbuf[pl.ds(0, LN)]
    alpha = ag[0]
    gamma = ag[1]
    a1 = 1.0 - alpha

    def _fill(t, carry):
        s_stg[pl.ds(t * LN, LN)] = jnp.zeros((LN,), jnp.int32)
        d_stg[pl.ds(t * LN, LN)] = jnp.zeros((LN,), jnp.int32) + DUMP
        w_stg[pl.ds(t * LN, LN)] = jnp.zeros((LN,), jnp.int32)
        return carry
    lax.fori_loop(0, NVEC, _fill, 0)

    def _zdeg(t, carry):
        degv[pl.ds(t * LN, LN)] = jnp.zeros((LN,), jnp.float32)
        return carry
    lax.fori_loop(0, ACCR // LN, _zdeg, 0)

    def _issue(sref, dref, wref, p, b, weighted):
        off = b * BLK
        pltpu.async_copy(sref.at[pl.ds(off, BLK)], sblk.at[p], bsem.at[p])
        pltpu.async_copy(dref.at[pl.ds(off, BLK)], dblk.at[p], bsem.at[p])
        if weighted:
            pltpu.async_copy(wref.at[pl.ds(off, BLK)], wblk.at[p],
                             bsem.at[p])

    def _scan_set(sref, dref, wref, wptr0, weighted):
        for p in range(2):
            _issue(sref, dref, wref, p, p, weighted)

        def _grp(g, wptr):
            for p in range(2):
                b = g * 2 + p

                @pl.when(b < NBLK)
                def _w():
                    for _ in range(3 if weighted else 2):
                        pltpu.make_async_copy(
                            sref.at[pl.ds(0, BLK)], sblk.at[p],
                            bsem.at[p]).wait()

                valid = b < NBLK

                def _vec(kk, wp):
                    d16 = dblk[p, pl.ds(kk * LN, LN)]
                    mask = jnp.logical_and(
                        jnp.logical_and(d16 >= lo, d16 < hi), valid)
                    mi = mask.astype(jnp.int32)
                    cnt = plsc.all_reduce_population_count(mask)[0]
                    dl16 = jnp.clip(d16 - lo, 0, RPW - 1)
                    ranks = plsc.cumsum(mi) - mi
                    wp2 = jnp.minimum(wp, CAPM - LN)
                    pos = wp2 + ranks
                    s16 = sblk[p, pl.ds(kk * LN, LN)]
                    plsc.store_scatter(s_stg, [pos], s16, mask=mask)
                    plsc.store_scatter(d_stg, [pos], dl16, mask=mask)
                    if weighted:
                        w16 = wblk[p, pl.ds(kk * LN, LN)]
                        plsc.store_scatter(w_stg, [pos], w16, mask=mask)
                    return wp + cnt
                wptr2 = lax.fori_loop(0, BLK // LN, _vec, wptr)

                @pl.when(b + 2 < NBLK)
                def _i():
                    _issue(sref, dref, wref, p, b + 2, weighted)
                wptr = wptr2
            return wptr
        return lax.fori_loop(0, (NBLK + 1) // 2, _grp, wptr0)

    cl = _scan_set(srcl, dstl, wgi, 0, weighted=False)
    cl = jnp.minimum(cl, CAPM)
    clp = jnp.minimum((cl + CH - 1) // CH * CH, CAPM)
    cm = _scan_set(srcg, dstg, wgi, clp, weighted=True)
    cm = jnp.minimum(cm, CAPM)

    lane0 = iota < 1

    def _deg(t, carry):
        base = t * LN
        dl16 = d_stg[pl.ds(base, LN)]
        for ii in range(LN):
            ridx = jnp.zeros((LN,), jnp.int32) + dl16[ii]
            m = jnp.logical_and(lane0, base + ii < cl)
            plsc.addupdate_scatter(degv, [ridx], ones16, mask=m)
        return carry
    lax.fori_loop(0, NVEC, _deg, 0)

    def _inv(t, carry):
        dv = degv[pl.ds(t * LN, LN)]
        degv[pl.ds(t * LN, LN)] = a1 / jnp.maximum(dv, 1.0)
        return carry
    lax.fori_loop(0, ACCR // LN, _inv, 0)

    a1g = a1 * gamma

    def _rw(t, carry):
        base = t * LN
        wv = plsc.bitcast(w_stg[pl.ds(base, LN)], jnp.float32)
        w_stg[pl.ds(base, LN)] = plsc.bitcast(wv * a1g, jnp.int32)
        return carry
    lax.fori_loop(0, NVEC, _rw, 0)

    cbuf[pl.ds(0, LN)] = jnp.where(
        iota < 1, cm, jnp.where(iota < 2, clp, 0))
    pltpu.sync_copy(cbuf, cnt_out.at[wid])
    pltpu.sync_copy(degv, inv_out.at[wid])
    pltpu.sync_copy(s_stg, rec_out.at[wid, pl.ds(0, CAPM)])
    pltpu.sync_copy(d_stg, rec_out.at[wid, pl.ds(CAPM, CAPM)])
    pltpu.sync_copy(w_stg, rec_out.at[wid, pl.ds(2 * CAPM, CAPM)])


@functools.partial(
    pl.kernel,
    out_type=jax.ShapeDtypeStruct((NP, C), jnp.float32),
    mesh=_mesh,
    scratch_types=[
        pltpu.VMEM((ACCR, C), jnp.float32),
        pltpu.VMEM((CAPM,), jnp.int32),
        pltpu.VMEM((CAPM,), jnp.int32),
        pltpu.VMEM((CAPM,), jnp.int32),
        pltpu.VMEM((NB, CH, C), jnp.float32),
        pltpu.VMEM((ACCR,), jnp.float32),
        pltpu.VMEM((16,), jnp.int32),
        pltpu.SemaphoreType.DMA((NB,)),
    ],
    compiler_params=_SC_PARAMS,
)
def _sc_aggregate(h_hbm, rec_hbm, cnt_hbm, inv_hbm,
                  out_hbm, acc, sbuf, dbuf, wbuf, rows, invdv, cntb, gsem):
    cid = lax.axis_index("c")
    sid = lax.axis_index("s")
    wid = sid * NC + cid
    iota = lax.iota(jnp.int32, LN)
    cols = [iota + j * LN for j in range(C // LN)]

    def _z(r, carry):
        for j in range(C // LN):
            acc[r, pl.ds(j * LN, LN)] = jnp.zeros((LN,), jnp.float32)
        return carry
    lax.fori_loop(0, ACCR, _z, 0)

    pltpu.sync_copy(cnt_hbm.at[wid], cntb)
    cv = cntb[pl.ds(0, LN)]
    cm = cv[0]
    clp = cv[1]
    nchl = clp // CH
    nch = (cm + CH - 1) // CH
    pltpu.sync_copy(rec_hbm.at[wid, pl.ds(0, CAPM)], sbuf)
    pltpu.sync_copy(rec_hbm.at[wid, pl.ds(CAPM, CAPM)], dbuf)
    pltpu.sync_copy(rec_hbm.at[wid, pl.ds(2 * CAPM, CAPM)], wbuf)
    pltpu.sync_copy(inv_hbm.at[wid], invdv)

    def _pipeline(c0, c1, weighted):
        span = c1 - c0
        ngroups = (span + LA + NB - 1) // NB

        def _grp(g, carry):
            for u in range(NB):
                i = g * NB + u

                @pl.when(i < span)
                def _issue():
                    pltpu.async_copy(
                        h_hbm.at[sbuf.at[pl.ds((c0 + i) * CH, CH)]],
                        rows.at[u], gsem.at[u])

                k = i - LA
                bu = (u + LA) % NB

                @pl.when(jnp.logical_and(k >= 0, k < span))
                def _process():
                    pltpu.make_async_copy(
                        h_hbm.at[sbuf.at[pl.ds(0, CH)]],
                        rows.at[bu], gsem.at[bu]).wait()

                    def _proc(kk, c3):
                        base = (c0 + k) * CH + kk * LN
                        dl16 = jnp.clip(dbuf[pl.ds(base, LN)], 0, ACCR - 1)
                        if weighted:
                            w16 = plsc.bitcast(wbuf[pl.ds(base, LN)],
                                               jnp.float32)
                        for ii in range(LN):
                            ridx = jnp.zeros((LN,), jnp.int32) + dl16[ii]
                            r = kk * LN + ii
                            for j in range(C // LN):
                                v = rows[bu, r, pl.ds(j * LN, LN)]
                                if weighted:
                                    v = v * w16[ii]
                                plsc.addupdate_scatter(
                                    acc, [ridx, cols[j]], v)
                        return c3
                    lax.fori_loop(0, CH // LN, _proc, 0)
            return carry
        lax.fori_loop(0, ngroups, _grp, 0)

    _pipeline(0, nchl, weighted=False)

    def _scale(r16, carry):
        s16 = invdv[pl.ds(r16 * LN, LN)]
        for ii in range(LN):
            s = s16[ii]
            r = r16 * LN + ii
            for j in range(C // LN):
                acc[r, pl.ds(j * LN, LN)] = acc[r, pl.ds(j * LN, LN)] * s
        return carry
    lax.fori_loop(0, RPW // LN, _scale, 0)

    _pipeline(nchl, nch, weighted=True)

    pltpu.sync_copy(acc.at[pl.ds(0, RPW)],
                    out_hbm.at[pl.ds(wid * RPW, RPW)])


def _pre_body(x_ref, w_ref, b_ref, sc_ref, h0_ref, ax0_ref):
    alpha = sc_ref[0, 0]
    h0 = jnp.dot(x_ref[...], w_ref[...],
                 preferred_element_type=jnp.float32) + b_ref[...]
    h0_ref[...] = h0
    ax0_ref[...] = alpha * h0


_tc_pre = pl.pallas_call(
    _pre_body,
    out_shape=(
        jax.ShapeDtypeStruct((NP, C), jnp.float32),
        jax.ShapeDtypeStruct((NP, C), jnp.float32),
    ),
    in_specs=[
        pl.BlockSpec(memory_space=pltpu.VMEM),
        pl.BlockSpec(memory_space=pltpu.VMEM),
        pl.BlockSpec(memory_space=pltpu.VMEM),
        pl.BlockSpec(memory_space=pltpu.SMEM),
    ],
)


def _layer_body(p_ref, ax0_ref, w_ref, b_ref, h_ref):
    hp = p_ref[...] + ax0_ref[...]
    h = jnp.dot(hp, w_ref[...], preferred_element_type=jnp.float32) + b_ref[...]
    h_ref[...] = jnp.maximum(h, 0.0)


_tc_layer = pl.pallas_call(
    _layer_body,
    out_shape=jax.ShapeDtypeStruct((NP, C), jnp.float32),
)


def _out_body(h_ref, w_ref, b_ref, o_ref):
    logits = jnp.dot(h_ref[:N], w_ref[...],
                     preferred_element_type=jnp.float32) + b_ref[...]
    m = jnp.max(logits, axis=1, keepdims=True)
    z = logits - m
    o_ref[...] = z - jnp.log(jnp.sum(jnp.exp(z), axis=1, keepdims=True))


_tc_out = pl.pallas_call(
    _out_body,
    out_shape=jax.ShapeDtypeStruct((N, NCLS), jnp.float32),
)


def kernel(x, edge_index, edge_index_global, edge_weight_global,
           W_in, b_in, W_layers, b_layers, W_out, b_out, alpha, gamma):
    pad = EPS - E
    sl2 = jnp.pad(edge_index[0], (0, pad))
    dl2 = jnp.pad(edge_index[1], (0, pad), constant_values=SENT)
    sg2 = jnp.pad(edge_index_global[0], (0, pad))
    dg2 = jnp.pad(edge_index_global[1], (0, pad), constant_values=SENT)
    wgi = lax.bitcast_convert_type(
        jnp.pad(edge_weight_global, (0, pad)), jnp.int32)
    scal16 = jnp.pad(jnp.stack([alpha, gamma]), (0, 14))
    rec, counts, invd = _sc_bucket(sl2, dl2, sg2, dg2, wgi, scal16)

    xp = jnp.pad(x, ((0, NP - N), (0, 0)))
    scal = jnp.stack([alpha, gamma]).reshape(1, 2)
    h0, ax0 = _tc_pre(xp, W_in, b_in.reshape(1, C), scal)

    h = h0
    for i in range(L - 2):
        part = _sc_aggregate(h, rec, counts, invd)
        h = _tc_layer(part, ax0, W_layers[i], b_layers[i].reshape(1, C))

    return _tc_out(h, W_out, b_out.reshape(1, NCLS))
